# baseline (speedup 1.0000x reference)
import functools

import jax
import jax.numpy as jnp
import numpy as np
from jax.experimental import pallas as pl
from jax.experimental.pallas import tpu as pltpu


def _bilinear_matrix(out_size: int, in_size: int) -> np.ndarray:
    W = np.zeros((out_size, in_size), dtype=np.float32)
    scale = in_size / out_size
    for i in range(out_size):
        src = (i + 0.5) * scale - 0.5
        src = max(src, 0.0)
        i0 = int(np.floor(src))
        i0 = min(i0, in_size - 1)
        i1 = min(i0 + 1, in_size - 1)
        lam = src - i0
        W[i, i0] += 1.0 - lam
        W[i, i1] += lam
    return W


@functools.lru_cache(maxsize=None)
def _interp_matrices(out_h: int, out_w: int, in_h: int, in_w: int):
    wh = jnp.asarray(_bilinear_matrix(out_h, in_h))
    wwt = jnp.asarray(np.ascontiguousarray(_bilinear_matrix(out_w, in_w).T))
    return wh, wwt


def _fused_kernel(ft_ref, fs_ref, wh_ref, wwt_ref, out_ref):
    eps = 1e-12
    ft = ft_ref[0].astype(jnp.float32)
    fs = fs_ref[0].astype(jnp.float32)

    s_tt = jnp.sum(ft * ft, axis=0)
    s_ss = jnp.sum(fs * fs, axis=0)
    s_ts = jnp.sum(ft * fs, axis=0)

    inv_t = 1.0 / jnp.maximum(jnp.sqrt(s_tt), eps)
    inv_s = 1.0 / jnp.maximum(jnp.sqrt(s_ss), eps)
    lm = 0.5 * (s_tt * inv_t * inv_t + s_ss * inv_s * inv_s) - s_ts * (inv_t * inv_s)

    tmp = jnp.dot(lm, wwt_ref[...], preferred_element_type=jnp.float32)
    out = jnp.dot(wh_ref[...], tmp, preferred_element_type=jnp.float32)
    out_ref[0, 0] = out


@jax.jit
def _forward(ft, fs, wh, wwt):
    B, C, Hf, Wf = ft.shape
    Hout, Wout = wh.shape[0], wwt.shape[1]
    HW = Hf * Wf

    itemsize = jnp.dtype(ft.dtype).itemsize
    cost = pl.CostEstimate(
        flops=int(B * (6 * C * HW + 12 * HW)
                  + 2 * B * (Hf * Wf * Wout + Hout * Hf * Wout)),
        transcendentals=int(2 * B * HW),
        bytes_accessed=int(2 * B * C * HW * itemsize + B * Hout * Wout * 4),
    )
    out = pl.pallas_call(
        _fused_kernel,
        out_shape=jax.ShapeDtypeStruct((B, 1, Hout, Wout), jnp.float32),
        grid=(B,),
        in_specs=[
            pl.BlockSpec((1, C, Hf, Wf), lambda b: (b, 0, 0, 0)),
            pl.BlockSpec((1, C, Hf, Wf), lambda b: (b, 0, 0, 0)),
            pl.BlockSpec((Hout, Hf), lambda b: (0, 0)),
            pl.BlockSpec((Wf, Wout), lambda b: (0, 0)),
        ],
        out_specs=pl.BlockSpec((1, 1, Hout, Wout), lambda b: (b, 0, 0, 0)),
        compiler_params=pltpu.CompilerParams(
            dimension_semantics=("parallel",),
            vmem_limit_bytes=100 << 20,
        ),
        cost_estimate=cost,
    )(ft, fs, wh, wwt)
    return out


def kernel(ft, fs):
    img_size = (32, 3, 256, 256)
    _, _, out_h, out_w = img_size
    _, _, Hf, Wf = ft.shape
    wh, wwt = _interp_matrices(int(out_h), int(out_w), int(Hf), int(Wf))
    return _forward(ft, fs, wh, wwt)


# speedup vs baseline: 5.8254x; 5.8254x over previous
import functools

import jax
import jax.numpy as jnp
import numpy as np
from jax.experimental import pallas as pl
from jax.experimental.pallas import tpu as pltpu


def _bilinear_matrix(out_size: int, in_size: int) -> np.ndarray:
    W = np.zeros((out_size, in_size), dtype=np.float32)
    scale = in_size / out_size
    for i in range(out_size):
        src = (i + 0.5) * scale - 0.5
        src = max(src, 0.0)
        i0 = int(np.floor(src))
        i0 = min(i0, in_size - 1)
        i1 = min(i0 + 1, in_size - 1)
        lam = src - i0
        W[i, i0] += 1.0 - lam
        W[i, i1] += lam
    return W


@functools.lru_cache(maxsize=None)
def _interp_matrices(out_h: int, out_w: int, in_h: int, in_w: int):
    wh = jnp.asarray(_bilinear_matrix(out_h, in_h))
    wwt = jnp.asarray(np.ascontiguousarray(_bilinear_matrix(out_w, in_w).T))
    return wh, wwt


def _fused_kernel(ft_ref, fs_ref, wh_ref, wwt_ref, out_ref):
    eps = 1e-12
    ft = ft_ref[0].astype(jnp.float32)
    fs = fs_ref[0].astype(jnp.float32)

    s_tt = jnp.sum(ft * ft, axis=-1)
    s_ss = jnp.sum(fs * fs, axis=-1)
    s_ts = jnp.sum(ft * fs, axis=-1)

    inv_t = 1.0 / jnp.maximum(jnp.sqrt(s_tt), eps)
    inv_s = 1.0 / jnp.maximum(jnp.sqrt(s_ss), eps)
    lm = 0.5 * (s_tt * inv_t * inv_t + s_ss * inv_s * inv_s) - s_ts * (inv_t * inv_s)

    tmp = jnp.dot(lm, wwt_ref[...], preferred_element_type=jnp.float32)
    out = jnp.dot(wh_ref[...], tmp, preferred_element_type=jnp.float32)
    out_ref[0, 0] = out


@jax.jit
def _forward(ft, fs, wh, wwt):
    B, C, Hf, Wf = ft.shape
    Hout, Wout = wh.shape[0], wwt.shape[1]
    HW = Hf * Wf

    ftt = jnp.transpose(ft, (0, 2, 3, 1))
    fst = jnp.transpose(fs, (0, 2, 3, 1))

    itemsize = jnp.dtype(ft.dtype).itemsize
    cost = pl.CostEstimate(
        flops=int(B * (6 * C * HW + 12 * HW)
                  + 2 * B * (Hf * Wf * Wout + Hout * Hf * Wout)),
        transcendentals=int(2 * B * HW),
        bytes_accessed=int(2 * B * C * HW * itemsize + B * Hout * Wout * 4),
    )
    out = pl.pallas_call(
        _fused_kernel,
        out_shape=jax.ShapeDtypeStruct((B, 1, Hout, Wout), jnp.float32),
        grid=(B,),
        in_specs=[
            pl.BlockSpec((1, Hf, Wf, C), lambda b: (b, 0, 0, 0)),
            pl.BlockSpec((1, Hf, Wf, C), lambda b: (b, 0, 0, 0)),
            pl.BlockSpec((Hout, Hf), lambda b: (0, 0)),
            pl.BlockSpec((Wf, Wout), lambda b: (0, 0)),
        ],
        out_specs=pl.BlockSpec((1, 1, Hout, Wout), lambda b: (b, 0, 0, 0)),
        compiler_params=pltpu.CompilerParams(
            dimension_semantics=("parallel",),
            vmem_limit_bytes=100 << 20,
        ),
        cost_estimate=cost,
    )(ftt, fst, wh, wwt)
    return out


def kernel(ft, fs):
    img_size = (32, 3, 256, 256)
    _, _, out_h, out_w = img_size
    _, _, Hf, Wf = ft.shape
    wh, wwt = _interp_matrices(int(out_h), int(out_w), int(Hf), int(Wf))
    return _forward(ft, fs, wh, wwt)


# speedup vs baseline: 6.6690x; 1.1448x over previous
import functools

import jax
import jax.numpy as jnp
import numpy as np
from jax.experimental import pallas as pl
from jax.experimental.pallas import tpu as pltpu


def _bilinear_matrix(out_size: int, in_size: int) -> np.ndarray:
    W = np.zeros((out_size, in_size), dtype=np.float32)
    scale = in_size / out_size
    for i in range(out_size):
        src = (i + 0.5) * scale - 0.5
        src = max(src, 0.0)
        i0 = int(np.floor(src))
        i0 = min(i0, in_size - 1)
        i1 = min(i0 + 1, in_size - 1)
        lam = src - i0
        W[i, i0] += 1.0 - lam
        W[i, i1] += lam
    return W


@functools.lru_cache(maxsize=None)
def _interp_matrices(out_h: int, out_w: int, in_h: int, in_w: int):
    wh = jnp.asarray(_bilinear_matrix(out_h, in_h))
    wwt = jnp.asarray(np.ascontiguousarray(_bilinear_matrix(out_w, in_w).T))
    return wh, wwt


def _fused_kernel(ft_ref, fs_ref, wh_ref, wwt_ref, out_ref, s_ref):
    eps = 1e-12
    ft = ft_ref[0].astype(jnp.float32)
    fs = fs_ref[0].astype(jnp.float32)

    s_ref[0] = jnp.sum(ft * ft, axis=-1)
    s_ref[1] = jnp.sum(fs * fs, axis=-1)
    s_ref[2] = jnp.sum(ft * fs, axis=-1)
    s_tt = s_ref[0]
    s_ss = s_ref[1]
    s_ts = s_ref[2]

    nt2 = jnp.maximum(s_tt, eps * eps)
    ns2 = jnp.maximum(s_ss, eps * eps)
    lm = 0.5 * (s_tt / nt2 + s_ss / ns2) - s_ts * jax.lax.rsqrt(nt2 * ns2)

    tmp = jnp.dot(lm, wwt_ref[...], preferred_element_type=jnp.float32)
    out = jnp.dot(wh_ref[...], tmp, preferred_element_type=jnp.float32)
    out_ref[0, 0] = out


@jax.jit
def _forward(ft, fs, wh, wwt):
    B, C, Hf, Wf = ft.shape
    Hout, Wout = wh.shape[0], wwt.shape[1]
    HW = Hf * Wf

    ftt = jnp.transpose(ft, (0, 2, 3, 1))
    fst = jnp.transpose(fs, (0, 2, 3, 1))

    itemsize = jnp.dtype(ft.dtype).itemsize
    cost = pl.CostEstimate(
        flops=int(B * (6 * C * HW + 12 * HW)
                  + 2 * B * (Hf * Wf * Wout + Hout * Hf * Wout)),
        transcendentals=int(2 * B * HW),
        bytes_accessed=int(2 * B * C * HW * itemsize + B * Hout * Wout * 4),
    )
    out = pl.pallas_call(
        _fused_kernel,
        out_shape=jax.ShapeDtypeStruct((B, 1, Hout, Wout), jnp.float32),
        grid=(B,),
        in_specs=[
            pl.BlockSpec((1, Hf, Wf, C), lambda b: (b, 0, 0, 0)),
            pl.BlockSpec((1, Hf, Wf, C), lambda b: (b, 0, 0, 0)),
            pl.BlockSpec((Hout, Hf), lambda b: (0, 0)),
            pl.BlockSpec((Wf, Wout), lambda b: (0, 0)),
        ],
        out_specs=pl.BlockSpec((1, 1, Hout, Wout), lambda b: (b, 0, 0, 0)),
        scratch_shapes=[pltpu.VMEM((3, Hf, Wf), jnp.float32)],
        compiler_params=pltpu.CompilerParams(
            dimension_semantics=("parallel",),
            vmem_limit_bytes=100 << 20,
        ),
        cost_estimate=cost,
    )(ftt, fst, wh, wwt)
    return out


def kernel(ft, fs):
    img_size = (32, 3, 256, 256)
    _, _, out_h, out_w = img_size
    _, _, Hf, Wf = ft.shape
    wh, wwt = _interp_matrices(int(out_h), int(out_w), int(Hf), int(Wf))
    return _forward(ft, fs, wh, wwt)


# speedup vs baseline: 7.3659x; 1.1045x over previous
import functools

import jax
import jax.numpy as jnp
import numpy as np
from jax.experimental import pallas as pl
from jax.experimental.pallas import tpu as pltpu


def _bilinear_matrix(out_size: int, in_size: int) -> np.ndarray:
    W = np.zeros((out_size, in_size), dtype=np.float32)
    scale = in_size / out_size
    for i in range(out_size):
        src = (i + 0.5) * scale - 0.5
        src = max(src, 0.0)
        i0 = int(np.floor(src))
        i0 = min(i0, in_size - 1)
        i1 = min(i0 + 1, in_size - 1)
        lam = src - i0
        W[i, i0] += 1.0 - lam
        W[i, i1] += lam
    return W


@functools.lru_cache(maxsize=None)
def _interp_matrices(out_h: int, out_w: int, in_h: int, in_w: int):
    wh = jnp.asarray(_bilinear_matrix(out_h, in_h))
    wwt = jnp.asarray(np.ascontiguousarray(_bilinear_matrix(out_w, in_w).T))
    return wh, wwt


def _fused_kernel(ft_ref, fs_ref, wh_ref, wwt_ref, out_ref, s_ref):
    eps = 1e-12
    for b in range(ft_ref.shape[0]):
        ft = ft_ref[b].astype(jnp.float32)
        fs = fs_ref[b].astype(jnp.float32)

        s_ref[0] = jnp.sum(ft * ft, axis=-1)
        s_ref[1] = jnp.sum(fs * fs, axis=-1)
        s_ref[2] = jnp.sum(ft * fs, axis=-1)
        s_tt = s_ref[0]
        s_ss = s_ref[1]
        s_ts = s_ref[2]

        nt2 = jnp.maximum(s_tt, eps * eps)
        ns2 = jnp.maximum(s_ss, eps * eps)
        lm = 0.5 * (s_tt / nt2 + s_ss / ns2) - s_ts * jax.lax.rsqrt(nt2 * ns2)

        tmp = jnp.dot(lm, wwt_ref[...], preferred_element_type=jnp.float32)
        out = jnp.dot(wh_ref[...], tmp, preferred_element_type=jnp.float32)
        out_ref[b, 0] = out


@jax.jit
def _forward(ft, fs, wh, wwt):
    B, C, Hf, Wf = ft.shape
    Hout, Wout = wh.shape[0], wwt.shape[1]
    HW = Hf * Wf

    ftt = jnp.transpose(ft, (0, 2, 3, 1))
    fst = jnp.transpose(fs, (0, 2, 3, 1))

    itemsize = jnp.dtype(ft.dtype).itemsize
    cost = pl.CostEstimate(
        flops=int(B * (6 * C * HW + 12 * HW)
                  + 2 * B * (Hf * Wf * Wout + Hout * Hf * Wout)),
        transcendentals=int(2 * B * HW),
        bytes_accessed=int(2 * B * C * HW * itemsize + B * Hout * Wout * 4),
    )
    bt = 2 if B % 2 == 0 else 1
    out = pl.pallas_call(
        _fused_kernel,
        out_shape=jax.ShapeDtypeStruct((B, 1, Hout, Wout), jnp.float32),
        grid=(B // bt,),
        in_specs=[
            pl.BlockSpec((bt, Hf, Wf, C), lambda b: (b, 0, 0, 0)),
            pl.BlockSpec((bt, Hf, Wf, C), lambda b: (b, 0, 0, 0)),
            pl.BlockSpec((Hout, Hf), lambda b: (0, 0)),
            pl.BlockSpec((Wf, Wout), lambda b: (0, 0)),
        ],
        out_specs=pl.BlockSpec((bt, 1, Hout, Wout), lambda b: (b, 0, 0, 0)),
        scratch_shapes=[pltpu.VMEM((3, Hf, Wf), jnp.float32)],
        compiler_params=pltpu.CompilerParams(
            dimension_semantics=("parallel",),
            vmem_limit_bytes=100 << 20,
        ),
        cost_estimate=cost,
    )(ftt, fst, wh, wwt)
    return out


def kernel(ft, fs):
    img_size = (32, 3, 256, 256)
    _, _, out_h, out_w = img_size
    _, _, Hf, Wf = ft.shape
    wh, wwt = _interp_matrices(int(out_h), int(out_w), int(Hf), int(Wf))
    return _forward(ft, fs, wh, wwt)
